# baseline (speedup 1.0000x reference)
import functools

import jax
import jax.numpy as jnp
from jax import lax
from jax.experimental import pallas as pl
from jax.experimental.pallas import tpu as pltpu
from jax.experimental.pallas import tpu_sc as plsc

N_FACTORS = 32
BATCH = 16384
TCOL = 128
CHUNK = 4
NBUF = 2


def _make_kernel(n_rows):
    info = plsc.get_sparse_core_info()
    nc, ns, nl = info.num_cores, info.num_subcores, info.num_lanes
    nw = nc * ns
    bpw = BATCH // nw
    ngrp = bpw // nl
    nslot = NBUF * CHUNK
    last_col = (n_rows // TCOL) * TCOL
    last_w = n_rows - last_col

    mesh = plsc.VectorSubcoreMesh(core_axis_name="c", subcore_axis_name="s")

    @functools.partial(
        pl.kernel,
        out_type=jax.ShapeDtypeStruct((BATCH,), jnp.float32),
        mesh=mesh,
        compiler_params=pltpu.CompilerParams(
            needs_layout_passes=False, use_tc_tiling_on_sc=True),
        scratch_types=[
            pltpu.VMEM((bpw + 2 * nl,), jnp.int32),
            pltpu.VMEM((bpw + 2 * nl,), jnp.int32),
            pltpu.VMEM((bpw,), jnp.float32),
            pltpu.VMEM((bpw,), jnp.float32),
            pltpu.VMEM((2 * N_FACTORS,), jnp.float32),
            pltpu.VMEM((N_FACTORS,), jnp.float32),
            pltpu.VMEM((N_FACTORS,), jnp.float32),
            pltpu.VMEM((N_FACTORS * nl,), jnp.float32),
            pltpu.VMEM((nl,), jnp.float32),
            pltpu.VMEM((nslot, N_FACTORS, TCOL), jnp.float32),
            pltpu.VMEM((nslot, N_FACTORS, TCOL), jnp.float32),
            pltpu.VMEM((N_FACTORS, last_w), jnp.float32),
            pltpu.VMEM((N_FACTORS, last_w), jnp.float32),
            pltpu.VMEM((bpw,), jnp.float32),
            pltpu.SemaphoreType.DMA,
            pltpu.SemaphoreType.DMA,
            pltpu.SemaphoreType.DMA,
            pltpu.SemaphoreType.DMA,
            pltpu.SemaphoreType.DMA,
        ],
    )
    def k(user_hbm, item_hbm, ctx0_hbm, ctx1_hbm, utabT_hbm, itabT_hbm,
          wc_hbm, bc_hbm, wo_hbm, wob_hbm, bo_hbm, out_hbm,
          idx_u, idx_i, ctx0_v, ctx1_v, wc_v, bc_v, wo_v, wob_v, bo_v,
          ubuf, ibuf, upart, ipart, out_v,
          sem_u0, sem_u1, sem_i0, sem_i1, sem_p):
        wid = lax.axis_index("s") * nc + lax.axis_index("c")
        base = wid * bpw
        sem_u = (sem_u0, sem_u1)
        sem_i = (sem_i0, sem_i1)

        pltpu.sync_copy(user_hbm.at[pl.ds(base, bpw)],
                        idx_u.at[pl.ds(0, bpw)])
        pltpu.sync_copy(item_hbm.at[pl.ds(base, bpw)],
                        idx_i.at[pl.ds(0, bpw)])
        pltpu.sync_copy(ctx0_hbm.at[pl.ds(base, bpw)], ctx0_v)
        pltpu.sync_copy(ctx1_hbm.at[pl.ds(base, bpw)], ctx1_v)
        pltpu.sync_copy(wc_hbm, wc_v)
        pltpu.sync_copy(bc_hbm, bc_v)
        pltpu.sync_copy(wo_hbm, wo_v)
        pltpu.sync_copy(wob_hbm, wob_v)
        pltpu.sync_copy(bo_hbm, bo_v)

        lanes = lax.iota(jnp.int32, nl)

        def allsum(x):
            for s in (8, 4, 2, 1):
                x = x + x.at[lanes ^ s].get(mode="promise_in_bounds")
            return x

        wo_lo = wo_v[pl.ds(0, nl)]
        wo_hi = wo_v[pl.ds(nl, nl)]
        wa = allsum(wc_v[pl.ds(0, nl)] * wo_lo + wc_v[pl.ds(nl, nl)] * wo_hi)
        wb = allsum(wc_v[pl.ds(2 * nl, nl)] * wo_lo
                    + wc_v[pl.ds(3 * nl, nl)] * wo_hi)
        const = (allsum(bc_v[pl.ds(0, nl)] * wo_lo
                        + bc_v[pl.ds(nl, nl)] * wo_hi)
                 + bo_v[...])

        onehots = [
            jnp.where(lanes == l, jnp.float32(1.0), jnp.float32(0.0))
            for l in range(nl)
        ]

        def fetch(tab_hbm, buf, sem, slot, idx_scalar):
            cb = (idx_scalar >> 7) << 7
            cb = jnp.minimum(cb, last_col - TCOL)
            cb = pl.multiple_of(cb, TCOL)
            pltpu.async_copy(tab_hbm.at[:, pl.ds(cb, TCOL)],
                             buf.at[slot], sem)

        def wait_tile(tab_hbm, buf, sem, slot):
            pltpu.make_async_copy(tab_hbm.at[:, pl.ds(0, TCOL)],
                                  buf.at[slot], sem).wait()

        cp_u = pltpu.async_copy(utabT_hbm.at[:, pl.ds(last_col, last_w)],
                                upart, sem_p)
        cp_i = pltpu.async_copy(itabT_hbm.at[:, pl.ds(last_col, last_w)],
                                ipart, sem_p)
        cp_u.wait()
        cp_i.wait()

        iv_u0 = idx_u[pl.ds(0, nl)]
        iv_i0 = idx_i[pl.ds(0, nl)]
        for j in range(CHUNK):
            fetch(utabT_hbm, ubuf, sem_u[0], j, iv_u0[j])
            fetch(itabT_hbm, ibuf, sem_i[0], j, iv_i0[j])

        def g_body(g, _):
            iv_u = idx_u[pl.ds(g * nl, nl)]
            iv_i = idx_i[pl.ds(g * nl, nl)]
            iv_u_nx = idx_u[pl.ds(g * nl + nl, nl)]
            iv_i_nx = idx_i[pl.ds(g * nl + nl, nl)]
            e_base = g * nl
            ctx0 = ctx0_v[pl.ds(e_base, nl)]
            ctx1 = ctx1_v[pl.ds(e_base, nl)]
            acc = ctx0 * wa + ctx1 * wb + const

            for kk in range(nl // CHUNK):
                par = kk % NBUF
                npar = (kk + 1) % NBUF
                for j in range(CHUNK):
                    l_nx = (kk + 1) * CHUNK + j
                    if l_nx < nl:
                        inu, ini = iv_u[l_nx], iv_i[l_nx]
                    else:
                        inu, ini = iv_u_nx[l_nx - nl], iv_i_nx[l_nx - nl]
                    e_nx = e_base + l_nx
                    inu = jnp.where(e_nx < bpw, inu, 0)
                    ini = jnp.where(e_nx < bpw, ini, 0)
                    slot = npar * CHUNK + j
                    fetch(utabT_hbm, ubuf, sem_u[npar], slot, inu)
                    fetch(itabT_hbm, ibuf, sem_i[npar], slot, ini)
                for j in range(CHUNK):
                    slot = par * CHUNK + j
                    wait_tile(utabT_hbm, ubuf, sem_u[par], slot)
                    wait_tile(itabT_hbm, ibuf, sem_i[par], slot)
                for j in range(CHUNK):
                    l = kk * CHUNK + j
                    slot = par * CHUNK + j
                    iu, ii = iv_u[l], iv_i[l]
                    cu = jnp.full((nl,), iu & (TCOL - 1), jnp.int32)
                    ci = jnp.full((nl,), ii & (TCOL - 1), jnp.int32)
                    pu = jnp.full(
                        (nl,),
                        jnp.clip(iu - last_col, 0, last_w - 1), jnp.int32)
                    pi = jnp.full(
                        (nl,),
                        jnp.clip(ii - last_col, 0, last_w - 1), jnp.int32)
                    sv = jnp.full((nl,), slot, jnp.int32)
                    u_lo = jnp.where(
                        iu >= last_col,
                        plsc.load_gather(upart, [lanes, pu]),
                        plsc.load_gather(ubuf, [sv, lanes, cu]))
                    u_hi = jnp.where(
                        iu >= last_col,
                        plsc.load_gather(upart, [lanes + nl, pu]),
                        plsc.load_gather(ubuf, [sv, lanes + nl, cu]))
                    v_lo = jnp.where(
                        ii >= last_col,
                        plsc.load_gather(ipart, [lanes, pi]),
                        plsc.load_gather(ibuf, [sv, lanes, ci]))
                    v_hi = jnp.where(
                        ii >= last_col,
                        plsc.load_gather(ipart, [lanes + nl, pi]),
                        plsc.load_gather(ibuf, [sv, lanes + nl, ci]))
                    s = allsum(u_lo * v_lo * wo_lo + u_hi * v_hi * wo_hi)
                    acc = acc + s * onehots[l]
            out_v[pl.ds(e_base, nl)] = acc
            return 0

        lax.fori_loop(0, ngrp, g_body, 0)

        for j in range(CHUNK):
            wait_tile(utabT_hbm, ubuf, sem_u[0], j)
            wait_tile(itabT_hbm, ibuf, sem_i[0], j)

        pltpu.sync_copy(out_v, out_hbm.at[pl.ds(base, bpw)])

    return k


def kernel(user, item, context, user_table, item_table, Wc, bc, Wo, bo):
    k = _make_kernel(user_table.shape[0])
    user_i = user.astype(jnp.int32)
    item_i = item.astype(jnp.int32)
    ctx0 = context[:, 0]
    ctx1 = context[:, 1]
    wc_flat = Wc.reshape(2 * N_FACTORS)
    wo_flat = Wo.reshape(N_FACTORS)
    wob_flat = jnp.broadcast_to(
        Wo.reshape(N_FACTORS, 1), (N_FACTORS, 16)).reshape(N_FACTORS * 16)
    bo_splat = jnp.broadcast_to(bo, (16,))
    return k(user_i, item_i, ctx0, ctx1, user_table.T, item_table.T,
             wc_flat, bc, wo_flat, wob_flat, bo_splat)
